# Initial kernel scaffold; baseline (speedup 1.0000x reference)
import jax
import jax.numpy as jnp
from jax.experimental import pallas as pl


def kernel(x, edge_index, weight, bias):
    raise NotImplementedError("write your pallas kernel here")


# speedup vs baseline: 23.3477x; 23.3477x over previous
import functools

import jax
import jax.numpy as jnp
from jax import lax
from jax.experimental import pallas as pl
from jax.experimental.pallas import tpu as pltpu
from jax.experimental.pallas import tpu_sc as plsc

N = 10000
E = 320000
D = 128

NCORES = 2
NSUB = 16
NW = NCORES * NSUB

CHUNK = 128
CPT = 80
EPT = CPT * CHUNK
EPAD = EPT * NW
NPAD = 10240
RPT = NPAD // NSUB


@functools.cache
def _mesh():
    return plsc.VectorSubcoreMesh(
        core_axis_name="c", subcore_axis_name="s",
        num_cores=NCORES, num_subcores=NSUB)


def _deg_body(colp, iota, ones, zrows, dd, idxs_v, iot_v, ones_v,
              rows_v, deg_sh):
    c = lax.axis_index("c")
    s = lax.axis_index("s")
    nck = RPT // CHUNK
    pltpu.sync_copy(ones, ones_v)
    pltpu.sync_copy(zrows, rows_v)
    pltpu.sync_copy(iota.at[pl.ds(s * 8, 8)], iot_v)
    for t in range(nck):
        pltpu.sync_copy(rows_v, deg_sh.at[iot_v.at[t]])
    plsc.subcore_barrier()

    tcb = c * (NSUB * CPT) + s * CPT
    pltpu.sync_copy(colp.at[pl.ds(tcb, CPT)], idxs_v)

    def _fire(j, _):
        pltpu.sync_copy(ones_v, deg_sh.at[idxs_v.at[j]], add=True)
        return 0

    lax.fori_loop(0, CPT, _fire, 0)
    plsc.subcore_barrier()

    for t in range(nck):
        pltpu.sync_copy(deg_sh.at[iot_v.at[t]], rows_v)
        pltpu.sync_copy(rows_v, dd.at[c, pl.ds(s * RPT + t * CHUNK, CHUNK)])


@functools.cache
def _deg_call():
    return pl.kernel(
        _deg_body,
        out_type=jax.ShapeDtypeStruct((NCORES, NPAD, D), jnp.float32),
        mesh=_mesh(),
        scratch_types=(
            pltpu.VMEM((CPT, CHUNK), jnp.int32),
            pltpu.VMEM((8, CHUNK), jnp.int32),
            pltpu.VMEM((CHUNK, D), jnp.float32),
            pltpu.VMEM((CHUNK, D), jnp.float32),
            pltpu.VMEM_SHARED((NPAD, D), jnp.float32),
        ),
    )


def _agg_body(g, rowp, colp, iota, zrows, ss, rows_v, ridx_v, cidx_v,
              iot_v, acc_sh):
    c = lax.axis_index("c")
    s = lax.axis_index("s")
    nck = RPT // CHUNK

    pltpu.sync_copy(zrows, rows_v)
    pltpu.sync_copy(iota.at[pl.ds(s * 8, 8)], iot_v)
    for t in range(nck):
        pltpu.sync_copy(rows_v, acc_sh.at[iot_v.at[t]])
    plsc.subcore_barrier()

    tcb = c * (NSUB * CPT) + s * CPT
    pltpu.sync_copy(rowp.at[pl.ds(tcb, CPT)], ridx_v)
    pltpu.sync_copy(colp.at[pl.ds(tcb, CPT)], cidx_v)

    def _step(j, _):
        pltpu.sync_copy(g.at[ridx_v.at[j]], rows_v)
        pltpu.sync_copy(rows_v, acc_sh.at[cidx_v.at[j]], add=True)
        return 0

    lax.fori_loop(0, CPT, _step, 0)
    plsc.subcore_barrier()

    for t in range(nck):
        rt = pl.ds(s * RPT + t * CHUNK, CHUNK)
        pltpu.sync_copy(acc_sh.at[iot_v.at[t]], rows_v)
        pltpu.sync_copy(rows_v, ss.at[c, rt])


@functools.cache
def _agg_call():
    return pl.kernel(
        _agg_body,
        out_type=jax.ShapeDtypeStruct((NCORES, NPAD, D), jnp.float32),
        mesh=_mesh(),
        scratch_types=(
            pltpu.VMEM((CHUNK, D), jnp.float32),
            pltpu.VMEM((CPT, CHUNK), jnp.int32),
            pltpu.VMEM((CPT, CHUNK), jnp.int32),
            pltpu.VMEM((8, CHUNK), jnp.int32),
            pltpu.VMEM_SHARED((NPAD, D), jnp.float32),
        ),
    )


_MMB = 1280


def _mm_body(x_ref, w_ref, p0_ref, p1_ref, h_ref, g_ref):
    h = jnp.dot(x_ref[...], w_ref[...], preferred_element_type=jnp.float32)
    deg = p0_ref[...] + p1_ref[...] + 1.0
    dinv = lax.rsqrt(deg)
    h_ref[...] = h
    g_ref[...] = h * dinv


_mm_call = pl.pallas_call(
    _mm_body,
    grid=(NPAD // _MMB,),
    in_specs=[
        pl.BlockSpec((_MMB, D), lambda i: (i, 0)),
        pl.BlockSpec((D, D), lambda i: (0, 0)),
        pl.BlockSpec((_MMB, 1), lambda i: (i, 0)),
        pl.BlockSpec((_MMB, 1), lambda i: (i, 0)),
    ],
    out_specs=[
        pl.BlockSpec((_MMB, D), lambda i: (i, 0)),
        pl.BlockSpec((_MMB, D), lambda i: (i, 0)),
    ],
    out_shape=[
        jax.ShapeDtypeStruct((NPAD, D), jnp.float32),
        jax.ShapeDtypeStruct((NPAD, D), jnp.float32),
    ],
)


_OB = 2000


def _out_body(s0_ref, s1_ref, h_ref, p0_ref, p1_ref, b_ref, o_ref):
    deg = p0_ref[...] + p1_ref[...] + 1.0
    dinv = lax.rsqrt(deg)
    ssum = s0_ref[...] + s1_ref[...]
    o_ref[...] = dinv * ssum + h_ref[...] / deg + b_ref[...]


_out_call = pl.pallas_call(
    _out_body,
    grid=(N // _OB,),
    in_specs=[
        pl.BlockSpec((_OB, D), lambda i: (i, 0)),
        pl.BlockSpec((_OB, D), lambda i: (i, 0)),
        pl.BlockSpec((_OB, D), lambda i: (i, 0)),
        pl.BlockSpec((_OB, 1), lambda i: (i, 0)),
        pl.BlockSpec((_OB, 1), lambda i: (i, 0)),
        pl.BlockSpec((1, D), lambda i: (0, 0)),
    ],
    out_specs=pl.BlockSpec((_OB, D), lambda i: (i, 0)),
    out_shape=jax.ShapeDtypeStruct((N, D), jnp.float32),
)


@jax.jit
def kernel(x, edge_index, weight, bias):
    row = edge_index[0]
    col = edge_index[1]

    pad = EPAD - E
    pad_rows = (jnp.arange(pad, dtype=jnp.int32) * 997) % N
    pad_cols = N + (jnp.arange(pad, dtype=jnp.int32) % (NPAD - N))
    rowp = jnp.concatenate([row, pad_rows]).reshape(EPAD // CHUNK, CHUNK)
    colp = jnp.concatenate([col, pad_cols]).reshape(EPAD // CHUNK, CHUNK)

    ones128 = jnp.ones((CHUNK, D), jnp.float32)
    zrows = jnp.zeros((CHUNK, D), jnp.float32)

    iota_n = jnp.pad(
        jnp.arange(NPAD, dtype=jnp.int32).reshape(NSUB, RPT // CHUNK, CHUNK),
        ((0, 0), (0, 8 - RPT // CHUNK), (0, 0))).reshape(NSUB * 8, CHUNK)
    dd = _deg_call()(colp, iota_n, ones128, zrows)
    p0 = dd[0, :, 0:1]
    p1 = dd[1, :, 0:1]

    xpad = jnp.concatenate(
        [x, jnp.zeros((NPAD - N, x.shape[1]), x.dtype)], axis=0)
    h, g = _mm_call(xpad, weight, p0, p1)

    ss = _agg_call()(g, rowp, colp, iota_n, zrows)
    s0 = ss[0]
    s1 = ss[1]

    out = _out_call(s0[:N], s1[:N], h[:N], p0[:N], p1[:N],
                    bias.reshape(1, D))
    return out


# speedup vs baseline: 27.1353x; 1.1622x over previous
import functools

import jax
import jax.numpy as jnp
from jax import lax
from jax.experimental import pallas as pl
from jax.experimental.pallas import tpu as pltpu
from jax.experimental.pallas import tpu_sc as plsc

N = 10000
E = 320000
D = 128

NCORES = 2
NSUB = 16
NW = NCORES * NSUB

CHUNK = 128
CPT = 80
EPT = CPT * CHUNK
EPAD = EPT * NW
NPAD = 10240
RPT = NPAD // NSUB


@functools.cache
def _mesh():
    return plsc.VectorSubcoreMesh(
        core_axis_name="c", subcore_axis_name="s",
        num_cores=NCORES, num_subcores=NSUB)


def _deg_body(colp, iota, ones, zrows, dd, idxs_v, iot_v, ones_v,
              rows_v, deg_sh):
    c = lax.axis_index("c")
    s = lax.axis_index("s")
    nck = RPT // CHUNK
    pltpu.sync_copy(ones, ones_v)
    pltpu.sync_copy(zrows, rows_v)
    pltpu.sync_copy(iota.at[pl.ds(s * 8, 8)], iot_v)
    for t in range(nck):
        pltpu.sync_copy(rows_v, deg_sh.at[iot_v.at[t]])
    plsc.subcore_barrier()

    tcb = c * (NSUB * CPT) + s * CPT
    pltpu.sync_copy(colp.at[pl.ds(tcb, CPT)], idxs_v)

    def _fire(j, _):
        pltpu.sync_copy(ones_v, deg_sh.at[idxs_v.at[j]], add=True)
        return 0

    lax.fori_loop(0, CPT, _fire, 0)
    plsc.subcore_barrier()

    for t in range(nck):
        pltpu.sync_copy(deg_sh.at[iot_v.at[t]], rows_v)
        pltpu.sync_copy(rows_v, dd.at[c, pl.ds(s * RPT + t * CHUNK, CHUNK)])


@functools.cache
def _deg_call():
    return pl.kernel(
        _deg_body,
        out_type=jax.ShapeDtypeStruct((NCORES, NPAD, D), jnp.float32),
        mesh=_mesh(),
        scratch_types=(
            pltpu.VMEM((CPT, CHUNK), jnp.int32),
            pltpu.VMEM((8, CHUNK), jnp.int32),
            pltpu.VMEM((CHUNK, D), jnp.float32),
            pltpu.VMEM((CHUNK, D), jnp.float32),
            pltpu.VMEM_SHARED((NPAD, D), jnp.float32),
        ),
    )


def _agg_body(g, rowp, colp, iota, zrows, ss, rows_v, r0_v, r1_v, cidx_v,
              iot_v, gs0, gs1, is0, is1, acc_sh):
    c = lax.axis_index("c")
    s = lax.axis_index("s")
    nck = RPT // CHUNK

    pltpu.sync_copy(zrows, rows_v.at[0])
    pltpu.sync_copy(iota.at[pl.ds(s * 8, 8)], iot_v)
    for t in range(nck):
        pltpu.sync_copy(rows_v.at[0], acc_sh.at[iot_v.at[t]])
    plsc.subcore_barrier()

    tcb = c * (NSUB * CPT) + s * CPT
    pltpu.sync_copy(colp.at[pl.ds(tcb, CPT)], cidx_v)

    def _idx_load(jc, dst, sem):
        pltpu.async_copy(rowp.at[jc], dst, sem)

    def _idx_wait(dst, sem):
        pltpu.make_async_copy(rowp.at[0], dst, sem).wait()

    def _gather(idxref, b, sem):
        pltpu.async_copy(g.at[idxref], rows_v.at[b], sem)

    def _gather_wait(b, sem):
        pltpu.make_async_copy(g.at[r0_v], rows_v.at[b], sem).wait()

    def _scat(j, b):
        pltpu.sync_copy(rows_v.at[b], acc_sh.at[cidx_v.at[j]], add=True)

    pltpu.sync_copy(rowp.at[tcb], r0_v)
    _idx_load(tcb + 1, r1_v, is1)
    _gather(r0_v, 0, gs0)

    def _pair(i, _):
        j0 = 2 * i
        j1 = j0 + 1
        _gather_wait(0, gs0)

        @pl.when(j0 + 2 < CPT)
        def _():
            _idx_load(tcb + j0 + 2, r0_v, is0)

        _idx_wait(r1_v, is1)
        _gather(r1_v, 1, gs1)
        _scat(j0, 0)
        _gather_wait(1, gs1)

        @pl.when(j1 + 2 < CPT)
        def _():
            _idx_load(tcb + j1 + 2, r1_v, is1)

        @pl.when(j0 + 2 < CPT)
        def _():
            _idx_wait(r0_v, is0)
            _gather(r0_v, 0, gs0)

        _scat(j1, 1)
        return 0

    lax.fori_loop(0, CPT // 2, _pair, 0)
    plsc.subcore_barrier()

    for t in range(nck):
        rt = pl.ds(s * RPT + t * CHUNK, CHUNK)
        pltpu.sync_copy(acc_sh.at[iot_v.at[t]], rows_v.at[0])
        pltpu.sync_copy(rows_v.at[0], ss.at[c, rt])


@functools.cache
def _agg_call():
    return pl.kernel(
        _agg_body,
        out_type=jax.ShapeDtypeStruct((NCORES, NPAD, D), jnp.float32),
        mesh=_mesh(),
        scratch_types=(
            pltpu.VMEM((2, CHUNK, D), jnp.float32),
            pltpu.VMEM((CHUNK,), jnp.int32),
            pltpu.VMEM((CHUNK,), jnp.int32),
            pltpu.VMEM((CPT, CHUNK), jnp.int32),
            pltpu.VMEM((8, CHUNK), jnp.int32),
            pltpu.SemaphoreType.DMA,
            pltpu.SemaphoreType.DMA,
            pltpu.SemaphoreType.DMA,
            pltpu.SemaphoreType.DMA,
            pltpu.VMEM_SHARED((NPAD, D), jnp.float32),
        ),
    )


_MMB = 1280


def _mm_body(x_ref, w_ref, p0_ref, p1_ref, h_ref, g_ref):
    h = jnp.dot(x_ref[...], w_ref[...], preferred_element_type=jnp.float32)
    deg = p0_ref[...] + p1_ref[...] + 1.0
    dinv = lax.rsqrt(deg)
    h_ref[...] = h
    g_ref[...] = h * dinv


_mm_call = pl.pallas_call(
    _mm_body,
    grid=(NPAD // _MMB,),
    in_specs=[
        pl.BlockSpec((_MMB, D), lambda i: (i, 0)),
        pl.BlockSpec((D, D), lambda i: (0, 0)),
        pl.BlockSpec((_MMB, 1), lambda i: (i, 0)),
        pl.BlockSpec((_MMB, 1), lambda i: (i, 0)),
    ],
    out_specs=[
        pl.BlockSpec((_MMB, D), lambda i: (i, 0)),
        pl.BlockSpec((_MMB, D), lambda i: (i, 0)),
    ],
    out_shape=[
        jax.ShapeDtypeStruct((NPAD, D), jnp.float32),
        jax.ShapeDtypeStruct((NPAD, D), jnp.float32),
    ],
)


_OB = 2000


def _out_body(s0_ref, s1_ref, h_ref, p0_ref, p1_ref, b_ref, o_ref):
    deg = p0_ref[...] + p1_ref[...] + 1.0
    dinv = lax.rsqrt(deg)
    ssum = s0_ref[...] + s1_ref[...]
    o_ref[...] = dinv * ssum + h_ref[...] / deg + b_ref[...]


_out_call = pl.pallas_call(
    _out_body,
    grid=(N // _OB,),
    in_specs=[
        pl.BlockSpec((_OB, D), lambda i: (i, 0)),
        pl.BlockSpec((_OB, D), lambda i: (i, 0)),
        pl.BlockSpec((_OB, D), lambda i: (i, 0)),
        pl.BlockSpec((_OB, 1), lambda i: (i, 0)),
        pl.BlockSpec((_OB, 1), lambda i: (i, 0)),
        pl.BlockSpec((1, D), lambda i: (0, 0)),
    ],
    out_specs=pl.BlockSpec((_OB, D), lambda i: (i, 0)),
    out_shape=jax.ShapeDtypeStruct((N, D), jnp.float32),
)


@jax.jit
def kernel(x, edge_index, weight, bias):
    row = edge_index[0]
    col = edge_index[1]

    pad = EPAD - E
    pad_rows = (jnp.arange(pad, dtype=jnp.int32) * 997) % N
    pad_cols = N + (jnp.arange(pad, dtype=jnp.int32) % (NPAD - N))
    rowp = jnp.concatenate([row, pad_rows]).reshape(EPAD // CHUNK, CHUNK)
    colp = jnp.concatenate([col, pad_cols]).reshape(EPAD // CHUNK, CHUNK)

    ones128 = jnp.ones((CHUNK, D), jnp.float32)
    zrows = jnp.zeros((CHUNK, D), jnp.float32)

    iota_n = jnp.pad(
        jnp.arange(NPAD, dtype=jnp.int32).reshape(NSUB, RPT // CHUNK, CHUNK),
        ((0, 0), (0, 8 - RPT // CHUNK), (0, 0))).reshape(NSUB * 8, CHUNK)
    dd = _deg_call()(colp, iota_n, ones128, zrows)
    p0 = dd[0, :, 0:1]
    p1 = dd[1, :, 0:1]

    xpad = jnp.concatenate(
        [x, jnp.zeros((NPAD - N, x.shape[1]), x.dtype)], axis=0)
    h, g = _mm_call(xpad, weight, p0, p1)

    ss = _agg_call()(g, rowp, colp, iota_n, zrows)
    s0 = ss[0]
    s1 = ss[1]

    out = _out_call(s0[:N], s1[:N], h[:N], p0[:N], p1[:N],
                    bias.reshape(1, D))
    return out
